# Initial kernel scaffold; baseline (speedup 1.0000x reference)
import jax
import jax.numpy as jnp
from jax.experimental import pallas as pl


def kernel(edge_attr, edge_index, node_attr, W1, b1, W2, b2, W3, b3):
    raise NotImplementedError("write your pallas kernel here")


# speedup vs baseline: 2.8501x; 2.8501x over previous
import functools

import jax
import jax.numpy as jnp
from jax import lax
from jax.experimental import pallas as pl
from jax.experimental.pallas import tpu as pltpu
from jax.experimental.pallas import tpu_sc as plsc

NC = 2
NS = 16
NW = NC * NS
LANES = 16

N_NODES = 10000
N_EDGES = 320000
F = 128

EPT = N_EDGES // NW
CHUNK = 80
NCHUNK_A = EPT // CHUNK
NCHUNK_C = 2 * EPT // CHUNK
ZROWS = 640
TBL = NS * ZROWS

_mesh = plsc.VectorSubcoreMesh(core_axis_name="c", subcore_axis_name="s")
_cparams = pltpu.CompilerParams(needs_layout_passes=False)


def _scatter_body(edge_hbm, src_hbm, z128_hbm, zflat_hbm,
                  sums_out, cnts_out,
                  idx_v, val_v, cnt_v, sums_sh):
    c = lax.axis_index("c")
    s = lax.axis_index("s")
    w = s * NC + c
    pltpu.sync_copy(z128_hbm, val_v)
    pltpu.sync_copy(zflat_hbm, cnt_v)

    @pl.loop(0, ZROWS // CHUNK)
    def _(k):
        pltpu.sync_copy(val_v, sums_sh.at[pl.ds(s * ZROWS + k * CHUNK, CHUNK)])

    pltpu.sync_copy(src_hbm.at[w], idx_v)
    plsc.subcore_barrier()
    ebase = w * EPT
    ones = jnp.ones((LANES,), jnp.float32)

    @pl.loop(0, NCHUNK_A)
    def _(j):
        pltpu.sync_copy(edge_hbm.at[pl.ds(ebase + j * CHUNK, CHUNK)], val_v)
        pltpu.sync_copy(val_v, sums_sh.at[idx_v.at[j]], add=True)

        @pl.loop(0, CHUNK // LANES)
        def _(k):
            idx16 = idx_v[j, pl.ds(k * LANES, LANES)]
            plsc.addupdate_scatter(cnt_v, [idx16], ones)

    plsc.subcore_barrier()

    @pl.loop(0, ZROWS // CHUNK)
    def _(k):
        r = s * ZROWS + k * CHUNK
        pltpu.sync_copy(sums_sh.at[pl.ds(r, CHUNK)], val_v)
        pltpu.sync_copy(val_v, sums_out.at[c].at[pl.ds(r, CHUNK)])

    pltpu.sync_copy(cnt_v, cnts_out.at[w])


_scatter_call = pl.kernel(
    _scatter_body,
    out_type=(
        jax.ShapeDtypeStruct((NC, TBL, F), jnp.float32),
        jax.ShapeDtypeStruct((NW, N_NODES), jnp.float32),
    ),
    mesh=_mesh,
    compiler_params=_cparams,
    scratch_types=[
        pltpu.VMEM((NCHUNK_A, CHUNK), jnp.int32),
        pltpu.VMEM((CHUNK, F), jnp.float32),
        pltpu.VMEM((N_NODES,), jnp.float32),
        pltpu.VMEM_SHARED((TBL, F), jnp.float32),
    ],
)


BN = 1024


def _dense_body(p_ref, c_ref, na_ref, w1_ref, b1_ref, w2_ref, b2_ref,
                w3_ref, b3_ref, h2_ref, st_ref):
    psum = p_ref[0] + p_ref[1]
    cnt = jnp.sum(c_ref[...], axis=0)[:, None]
    agg = psum / jnp.maximum(cnt, 1.0)
    x = jnp.concatenate([na_ref[...], agg], axis=1)
    h = jax.nn.sigmoid(
        jnp.dot(x, w1_ref[...], preferred_element_type=jnp.float32) + b1_ref[...])
    x2 = jnp.concatenate([h, agg], axis=1)
    h2 = jax.nn.sigmoid(
        jnp.dot(x2, w2_ref[...], preferred_element_type=jnp.float32) + b2_ref[...])
    h2_ref[...] = h2
    st_ref[...] = (
        jnp.dot(h2, w3_ref[...], preferred_element_type=jnp.float32) + b3_ref[...])


_dense_call = pl.pallas_call(
    _dense_body,
    grid=((N_NODES + BN - 1) // BN,),
    in_specs=[
        pl.BlockSpec((NC, BN, F), lambda i: (0, i, 0)),
        pl.BlockSpec((NW, BN), lambda i: (0, i)),
        pl.BlockSpec((BN, F), lambda i: (i, 0)),
        pl.BlockSpec((2 * F, F), lambda i: (0, 0)),
        pl.BlockSpec((1, F), lambda i: (0, 0)),
        pl.BlockSpec((2 * F, F), lambda i: (0, 0)),
        pl.BlockSpec((1, F), lambda i: (0, 0)),
        pl.BlockSpec((F, 8), lambda i: (0, 0)),
        pl.BlockSpec((1, 8), lambda i: (0, 0)),
    ],
    out_specs=[
        pl.BlockSpec((BN, F), lambda i: (i, 0)),
        pl.BlockSpec((BN, 8), lambda i: (i, 0)),
    ],
    out_shape=[
        jax.ShapeDtypeStruct((N_NODES, F), jnp.float32),
        jax.ShapeDtypeStruct((N_NODES, 8), jnp.float32),
    ],
)


def _gather_body(h2_hbm, idx2_hbm, src_hbm, dst_hbm, s_hbm, t_hbm,
                 emb_out, log_out,
                 idx2_v, gb_v, sv_v, tv_v, srcv, dstv, logv):
    c = lax.axis_index("c")
    s = lax.axis_index("s")
    w = s * NC + c
    pltpu.sync_copy(idx2_hbm.at[w], idx2_v)
    pltpu.sync_copy(src_hbm.at[w], srcv)
    pltpu.sync_copy(dst_hbm.at[w], dstv)
    pltpu.sync_copy(s_hbm, sv_v)
    pltpu.sync_copy(t_hbm, tv_v)
    rbase = w * 2 * EPT

    @pl.loop(0, NCHUNK_C)
    def _(j):
        pltpu.sync_copy(h2_hbm.at[idx2_v.at[j]], gb_v)
        pltpu.sync_copy(gb_v, emb_out.at[pl.ds(rbase + j * CHUNK, CHUNK)])

    @pl.loop(0, EPT // LANES)
    def _(i):
        si = srcv[pl.ds(i * LANES, LANES)]
        di = dstv[pl.ds(i * LANES, LANES)]
        vs = plsc.load_gather(sv_v, [si])
        vt = plsc.load_gather(tv_v, [di])
        logv[pl.ds(i * LANES, LANES)] = vs + vt

    pltpu.sync_copy(logv, log_out.at[pl.ds(w * EPT, EPT)])


_gather_call = pl.kernel(
    _gather_body,
    out_type=(
        jax.ShapeDtypeStruct((2 * N_EDGES, F), jnp.float32),
        jax.ShapeDtypeStruct((N_EDGES,), jnp.float32),
    ),
    mesh=_mesh,
    compiler_params=pltpu.CompilerParams(needs_layout_passes=False),
    scratch_types=[
        pltpu.VMEM((NCHUNK_C, CHUNK), jnp.int32),
        pltpu.VMEM((CHUNK, F), jnp.float32),
        pltpu.VMEM((N_NODES,), jnp.float32),
        pltpu.VMEM((N_NODES,), jnp.float32),
        pltpu.VMEM((EPT,), jnp.int32),
        pltpu.VMEM((EPT,), jnp.int32),
        pltpu.VMEM((EPT,), jnp.float32),
    ],
)


def kernel(edge_attr, edge_index, node_attr, W1, b1, W2, b2, W3, b3):
    src = edge_index[0, :].reshape(-1)
    dst = edge_index[1, :].reshape(-1)
    src3 = src.reshape(NW, NCHUNK_A, CHUNK)
    srcr = src.reshape(NW, EPT)
    dstr = dst.reshape(NW, EPT)
    idx2 = jnp.stack([src, dst], axis=1).reshape(NW, NCHUNK_C, CHUNK)
    z128 = jnp.zeros((CHUNK, F), jnp.float32)
    zflat = jnp.zeros((N_NODES,), jnp.float32)

    sums, cnts = _scatter_call(edge_attr, src3, z128, zflat)

    w3r = jnp.concatenate([W3[:F], W3[F:]], axis=1)
    w3r = jnp.pad(w3r, ((0, 0), (0, 6)))
    b3r = jnp.zeros((1, 8), jnp.float32).at[0, 0].set(b3[0])
    h2, st = _dense_call(sums, cnts, node_attr, W1, b1.reshape(1, F),
                         W2, b2.reshape(1, F), w3r, b3r)

    emb2, logits = _gather_call(h2, idx2, srcr, dstr,
                                st[:, 0], st[:, 1])
    return logits, emb2.reshape(N_EDGES, 2 * F)


# speedup vs baseline: 3.0658x; 1.0757x over previous
import functools

import jax
import jax.numpy as jnp
from jax import lax
from jax.experimental import pallas as pl
from jax.experimental.pallas import tpu as pltpu
from jax.experimental.pallas import tpu_sc as plsc

NC = 2
NS = 16
NW = NC * NS
LANES = 16

N_NODES = 10000
N_EDGES = 320000
F = 128

EPT = N_EDGES // NW
CHUNK = 80
NCHUNK_A = EPT // CHUNK
NCHUNK_C = 2 * EPT // CHUNK
ZROWS = 640
TBL = NS * ZROWS

_mesh = plsc.VectorSubcoreMesh(core_axis_name="c", subcore_axis_name="s")
_cparams = pltpu.CompilerParams(needs_layout_passes=False)


def _scatter_body(edge_hbm, src_hbm, z128_hbm, zflat_hbm,
                  sums_out, cnts_out,
                  idx_v, val_v, val_b, cnt_v, sums_sh, sem0, sem1):
    c = lax.axis_index("c")
    s = lax.axis_index("s")
    w = s * NC + c
    pltpu.sync_copy(z128_hbm, val_v)
    pltpu.sync_copy(zflat_hbm, cnt_v)

    @pl.loop(0, ZROWS // CHUNK)
    def _(k):
        pltpu.sync_copy(val_v, sums_sh.at[pl.ds(s * ZROWS + k * CHUNK, CHUNK)])

    pltpu.sync_copy(src_hbm.at[w], idx_v)
    plsc.subcore_barrier()
    ebase = w * EPT
    ones = jnp.ones((LANES,), jnp.float32)

    pltpu.async_copy(edge_hbm.at[pl.ds(ebase, CHUNK)], val_v, sem0)

    @pl.loop(0, NCHUNK_A - 1, step=2)
    def _(j):
        pltpu.make_async_copy(edge_hbm.at[pl.ds(0, CHUNK)], val_v, sem0).wait()
        pltpu.async_copy(
            edge_hbm.at[pl.ds(ebase + (j + 1) * CHUNK, CHUNK)], val_b, sem1)
        pltpu.sync_copy(val_v, sums_sh.at[idx_v.at[j]], add=True)
        pltpu.make_async_copy(edge_hbm.at[pl.ds(0, CHUNK)], val_b, sem1).wait()
        pltpu.async_copy(
            edge_hbm.at[pl.ds(ebase + (j + 2) * CHUNK, CHUNK)], val_v, sem0)
        pltpu.sync_copy(val_b, sums_sh.at[idx_v.at[j + 1]], add=True)

    jl = NCHUNK_A - 1
    pltpu.make_async_copy(edge_hbm.at[pl.ds(0, CHUNK)], val_v, sem0).wait()
    pltpu.sync_copy(val_v, sums_sh.at[idx_v.at[jl]], add=True)

    @pl.loop(0, NCHUNK_A)
    def _(j):
        @pl.loop(0, CHUNK // LANES)
        def _(k):
            idx16 = idx_v[j, pl.ds(k * LANES, LANES)]
            plsc.addupdate_scatter(cnt_v, [idx16], ones)

    plsc.subcore_barrier()

    @pl.loop(0, ZROWS // CHUNK)
    def _(k):
        r = s * ZROWS + k * CHUNK
        pltpu.sync_copy(sums_sh.at[pl.ds(r, CHUNK)], val_v)
        pltpu.sync_copy(val_v, sums_out.at[c].at[pl.ds(r, CHUNK)])

    pltpu.sync_copy(cnt_v, cnts_out.at[w])


_scatter_call = pl.kernel(
    _scatter_body,
    out_type=(
        jax.ShapeDtypeStruct((NC, TBL, F), jnp.float32),
        jax.ShapeDtypeStruct((NW, N_NODES), jnp.float32),
    ),
    mesh=_mesh,
    compiler_params=_cparams,
    scratch_types=[
        pltpu.VMEM((NCHUNK_A, CHUNK), jnp.int32),
        pltpu.VMEM((CHUNK, F), jnp.float32),
        pltpu.VMEM((CHUNK, F), jnp.float32),
        pltpu.VMEM((N_NODES,), jnp.float32),
        pltpu.VMEM_SHARED((TBL, F), jnp.float32),
        pltpu.SemaphoreType.DMA,
        pltpu.SemaphoreType.DMA,
    ],
)


BN = 1024


def _dense_body(p_ref, c_ref, na_ref, w1_ref, b1_ref, w2_ref, b2_ref,
                w3_ref, b3_ref, h2_ref, st_ref):
    psum = p_ref[0] + p_ref[1]
    cnt = jnp.sum(c_ref[...], axis=0)[:, None]
    agg = psum / jnp.maximum(cnt, 1.0)
    x = jnp.concatenate([na_ref[...], agg], axis=1)
    h = jax.nn.sigmoid(
        jnp.dot(x, w1_ref[...], preferred_element_type=jnp.float32) + b1_ref[...])
    x2 = jnp.concatenate([h, agg], axis=1)
    h2 = jax.nn.sigmoid(
        jnp.dot(x2, w2_ref[...], preferred_element_type=jnp.float32) + b2_ref[...])
    h2_ref[...] = h2
    st_ref[...] = (
        jnp.dot(h2, w3_ref[...], preferred_element_type=jnp.float32) + b3_ref[...])


_dense_call = pl.pallas_call(
    _dense_body,
    grid=((N_NODES + BN - 1) // BN,),
    in_specs=[
        pl.BlockSpec((NC, BN, F), lambda i: (0, i, 0)),
        pl.BlockSpec((NW, BN), lambda i: (0, i)),
        pl.BlockSpec((BN, F), lambda i: (i, 0)),
        pl.BlockSpec((2 * F, F), lambda i: (0, 0)),
        pl.BlockSpec((1, F), lambda i: (0, 0)),
        pl.BlockSpec((2 * F, F), lambda i: (0, 0)),
        pl.BlockSpec((1, F), lambda i: (0, 0)),
        pl.BlockSpec((F, 8), lambda i: (0, 0)),
        pl.BlockSpec((1, 8), lambda i: (0, 0)),
    ],
    out_specs=[
        pl.BlockSpec((BN, F), lambda i: (i, 0)),
        pl.BlockSpec((BN, 8), lambda i: (i, 0)),
    ],
    out_shape=[
        jax.ShapeDtypeStruct((N_NODES, F), jnp.float32),
        jax.ShapeDtypeStruct((N_NODES, 8), jnp.float32),
    ],
)


def _gather_body(h2_hbm, idx2_hbm, src_hbm, dst_hbm, s_hbm, t_hbm,
                 emb_out, log_out,
                 idx2_v, gb_v, gb_b, sv_v, tv_v, srcv, dstv, logv, gsem0, gsem1):
    c = lax.axis_index("c")
    s = lax.axis_index("s")
    w = s * NC + c
    pltpu.sync_copy(idx2_hbm.at[w], idx2_v)
    pltpu.sync_copy(src_hbm.at[w], srcv)
    pltpu.sync_copy(dst_hbm.at[w], dstv)
    pltpu.sync_copy(s_hbm, sv_v)
    pltpu.sync_copy(t_hbm, tv_v)
    rbase = w * 2 * EPT

    pltpu.async_copy(h2_hbm.at[idx2_v.at[0]], gb_v, gsem0)

    @pl.loop(0, NCHUNK_C, step=2)
    def _(j):
        pltpu.make_async_copy(h2_hbm.at[pl.ds(0, CHUNK)], gb_v, gsem0).wait()
        pltpu.async_copy(h2_hbm.at[idx2_v.at[j + 1]], gb_b, gsem1)
        pltpu.sync_copy(gb_v, emb_out.at[pl.ds(rbase + j * CHUNK, CHUNK)])
        pltpu.make_async_copy(h2_hbm.at[pl.ds(0, CHUNK)], gb_b, gsem1).wait()

        @pl.when(j + 2 < NCHUNK_C)
        def _():
            pltpu.async_copy(h2_hbm.at[idx2_v.at[j + 2]], gb_v, gsem0)

        pltpu.sync_copy(gb_b, emb_out.at[pl.ds(rbase + (j + 1) * CHUNK, CHUNK)])

    @pl.loop(0, EPT // LANES)
    def _(i):
        si = srcv[pl.ds(i * LANES, LANES)]
        di = dstv[pl.ds(i * LANES, LANES)]
        vs = plsc.load_gather(sv_v, [si])
        vt = plsc.load_gather(tv_v, [di])
        logv[pl.ds(i * LANES, LANES)] = vs + vt

    pltpu.sync_copy(logv, log_out.at[pl.ds(w * EPT, EPT)])


_gather_call = pl.kernel(
    _gather_body,
    out_type=(
        jax.ShapeDtypeStruct((2 * N_EDGES, F), jnp.float32),
        jax.ShapeDtypeStruct((N_EDGES,), jnp.float32),
    ),
    mesh=_mesh,
    compiler_params=pltpu.CompilerParams(needs_layout_passes=False),
    scratch_types=[
        pltpu.VMEM((NCHUNK_C, CHUNK), jnp.int32),
        pltpu.VMEM((CHUNK, F), jnp.float32),
        pltpu.VMEM((CHUNK, F), jnp.float32),
        pltpu.VMEM((N_NODES,), jnp.float32),
        pltpu.VMEM((N_NODES,), jnp.float32),
        pltpu.VMEM((EPT,), jnp.int32),
        pltpu.VMEM((EPT,), jnp.int32),
        pltpu.VMEM((EPT,), jnp.float32),
        pltpu.SemaphoreType.DMA,
        pltpu.SemaphoreType.DMA,
    ],
)


def kernel(edge_attr, edge_index, node_attr, W1, b1, W2, b2, W3, b3):
    src = edge_index[0, :].reshape(-1)
    dst = edge_index[1, :].reshape(-1)
    src3 = src.reshape(NW, NCHUNK_A, CHUNK)
    srcr = src.reshape(NW, EPT)
    dstr = dst.reshape(NW, EPT)
    idx2 = jnp.stack([src, dst], axis=1).reshape(NW, NCHUNK_C, CHUNK)
    z128 = jnp.zeros((CHUNK, F), jnp.float32)
    zflat = jnp.zeros((N_NODES,), jnp.float32)

    sums, cnts = _scatter_call(edge_attr, src3, z128, zflat)

    w3r = jnp.concatenate([W3[:F], W3[F:]], axis=1)
    w3r = jnp.pad(w3r, ((0, 0), (0, 6)))
    b3r = jnp.zeros((1, 8), jnp.float32).at[0, 0].set(b3[0])
    h2, st = _dense_call(sums, cnts, node_attr, W1, b1.reshape(1, F),
                         W2, b2.reshape(1, F), w3r, b3r)

    emb2, logits = _gather_call(h2, idx2, srcr, dstr,
                                st[:, 0], st[:, 1])
    return logits, emb2.reshape(N_EDGES, 2 * F)


# speedup vs baseline: 6.5711x; 2.1434x over previous
import jax
import jax.numpy as jnp
from jax import lax
from jax.experimental import pallas as pl
from jax.experimental.pallas import tpu as pltpu
from jax.experimental.pallas import tpu_sc as plsc

NC = 2
NS = 16
NW = NC * NS
LANES = 16

N_NODES = 10000
N_EDGES = 320000
F = 128

EPT = N_EDGES // NW
CHUNK = 80
NCHUNK_A = EPT // CHUNK
ZROWS = 640
TBL = NS * ZROWS

_mesh = plsc.VectorSubcoreMesh(core_axis_name="c", subcore_axis_name="s")
_cparams = pltpu.CompilerParams(needs_layout_passes=False)


def _scatter_body(edge_hbm, src_hbm, z128_hbm, zflat_hbm,
                  sums_out, cnts_out,
                  idx_v, val_v, val_b, cnt_v, sums_sh, sem0, sem1):
    c = lax.axis_index("c")
    s = lax.axis_index("s")
    w = s * NC + c
    pltpu.sync_copy(z128_hbm, val_v)
    pltpu.sync_copy(zflat_hbm, cnt_v)

    @pl.loop(0, ZROWS // CHUNK)
    def _(k):
        pltpu.sync_copy(val_v, sums_sh.at[pl.ds(s * ZROWS + k * CHUNK, CHUNK)])

    pltpu.sync_copy(src_hbm.at[w], idx_v)
    plsc.subcore_barrier()
    ebase = w * EPT
    ones = jnp.ones((LANES,), jnp.float32)

    pltpu.async_copy(edge_hbm.at[pl.ds(ebase, CHUNK)], val_v, sem0)

    @pl.loop(0, NCHUNK_A - 1, step=2)
    def _(j):
        pltpu.make_async_copy(edge_hbm.at[pl.ds(0, CHUNK)], val_v, sem0).wait()
        pltpu.async_copy(
            edge_hbm.at[pl.ds(ebase + (j + 1) * CHUNK, CHUNK)], val_b, sem1)
        pltpu.sync_copy(val_v, sums_sh.at[idx_v.at[j]], add=True)
        pltpu.make_async_copy(edge_hbm.at[pl.ds(0, CHUNK)], val_b, sem1).wait()
        pltpu.async_copy(
            edge_hbm.at[pl.ds(ebase + (j + 2) * CHUNK, CHUNK)], val_v, sem0)
        pltpu.sync_copy(val_b, sums_sh.at[idx_v.at[j + 1]], add=True)

    jl = NCHUNK_A - 1
    pltpu.make_async_copy(edge_hbm.at[pl.ds(0, CHUNK)], val_v, sem0).wait()
    pltpu.sync_copy(val_v, sums_sh.at[idx_v.at[jl]], add=True)

    @pl.loop(0, NCHUNK_A)
    def _(j):
        @pl.loop(0, CHUNK // LANES)
        def _(k):
            idx16 = idx_v[j, pl.ds(k * LANES, LANES)]
            plsc.addupdate_scatter(cnt_v, [idx16], ones)

    plsc.subcore_barrier()

    @pl.loop(0, ZROWS // CHUNK)
    def _(k):
        r = s * ZROWS + k * CHUNK
        pltpu.sync_copy(sums_sh.at[pl.ds(r, CHUNK)], val_v)
        pltpu.sync_copy(val_v, sums_out.at[c].at[pl.ds(r, CHUNK)])

    pltpu.sync_copy(cnt_v, cnts_out.at[w])


_scatter_call = pl.kernel(
    _scatter_body,
    out_type=(
        jax.ShapeDtypeStruct((NC, TBL, F), jnp.float32),
        jax.ShapeDtypeStruct((NW, N_NODES), jnp.float32),
    ),
    mesh=_mesh,
    compiler_params=_cparams,
    scratch_types=[
        pltpu.VMEM((NCHUNK_A, CHUNK), jnp.int32),
        pltpu.VMEM((CHUNK, F), jnp.float32),
        pltpu.VMEM((CHUNK, F), jnp.float32),
        pltpu.VMEM((N_NODES,), jnp.float32),
        pltpu.VMEM_SHARED((TBL, F), jnp.float32),
        pltpu.SemaphoreType.DMA,
        pltpu.SemaphoreType.DMA,
    ],
)


BN = 1024


def _dense_body(p_ref, c_ref, na_ref, w1_ref, b1_ref, w2_ref, b2_ref,
                w3_ref, b3_ref, h2_ref, st_ref):
    psum = p_ref[0] + p_ref[1]
    cnt = jnp.sum(c_ref[...], axis=0)[:, None]
    agg = psum / jnp.maximum(cnt, 1.0)
    x = jnp.concatenate([na_ref[...], agg], axis=1)
    h = jax.nn.sigmoid(
        jnp.dot(x, w1_ref[...], preferred_element_type=jnp.float32) + b1_ref[...])
    x2 = jnp.concatenate([h, agg], axis=1)
    h2 = jax.nn.sigmoid(
        jnp.dot(x2, w2_ref[...], preferred_element_type=jnp.float32) + b2_ref[...])
    h2_ref[...] = h2
    st_ref[...] = (
        jnp.dot(h2, w3_ref[...], preferred_element_type=jnp.float32) + b3_ref[...])


_dense_call = pl.pallas_call(
    _dense_body,
    grid=((N_NODES + BN - 1) // BN,),
    in_specs=[
        pl.BlockSpec((NC, BN, F), lambda i: (0, i, 0)),
        pl.BlockSpec((NW, BN), lambda i: (0, i)),
        pl.BlockSpec((BN, F), lambda i: (i, 0)),
        pl.BlockSpec((2 * F, F), lambda i: (0, 0)),
        pl.BlockSpec((1, F), lambda i: (0, 0)),
        pl.BlockSpec((2 * F, F), lambda i: (0, 0)),
        pl.BlockSpec((1, F), lambda i: (0, 0)),
        pl.BlockSpec((F, 8), lambda i: (0, 0)),
        pl.BlockSpec((1, 8), lambda i: (0, 0)),
    ],
    out_specs=[
        pl.BlockSpec((BN, F), lambda i: (i, 0)),
        pl.BlockSpec((BN, 8), lambda i: (i, 0)),
    ],
    out_shape=[
        jax.ShapeDtypeStruct((N_NODES, F), jnp.float32),
        jax.ShapeDtypeStruct((N_NODES, 8), jnp.float32),
    ],
)


def _gather_body(h2_hbm, src_hbm, dst_hbm, s_hbm, t_hbm,
                 emb_out, log_out,
                 ga0, gb0, ga1, gb1, sv_v, tv_v, srcv, dstv, logv,
                 semA0, semB0, semA1, semB1):
    c = lax.axis_index("c")
    s = lax.axis_index("s")
    w = s * NC + c
    pltpu.sync_copy(src_hbm.at[w], srcv)
    pltpu.sync_copy(dst_hbm.at[w], dstv)
    pltpu.sync_copy(s_hbm, sv_v)
    pltpu.sync_copy(t_hbm, tv_v)
    rbase = w * EPT
    dummy = h2_hbm.at[pl.ds(0, CHUNK)]

    pltpu.async_copy(h2_hbm.at[srcv.at[pl.ds(0, CHUNK)]], ga0, semA0)
    pltpu.async_copy(h2_hbm.at[dstv.at[pl.ds(0, CHUNK)]], gb0, semB0)

    @pl.loop(0, NCHUNK_A - 1, step=2)
    def _(j):
        r0 = rbase + j * CHUNK
        r1 = r0 + CHUNK
        pltpu.make_async_copy(dummy, ga0, semA0).wait()
        pltpu.async_copy(h2_hbm.at[srcv.at[pl.ds((j + 1) * CHUNK, CHUNK)]],
                         ga1, semA1)
        pltpu.sync_copy(ga0, emb_out.at[pl.ds(r0, CHUNK), pl.ds(0, F)])
        pltpu.make_async_copy(dummy, gb0, semB0).wait()
        pltpu.async_copy(h2_hbm.at[dstv.at[pl.ds((j + 1) * CHUNK, CHUNK)]],
                         gb1, semB1)
        pltpu.sync_copy(gb0, emb_out.at[pl.ds(r0, CHUNK), pl.ds(F, F)])
        pltpu.make_async_copy(dummy, ga1, semA1).wait()
        pltpu.async_copy(h2_hbm.at[srcv.at[pl.ds((j + 2) * CHUNK, CHUNK)]],
                         ga0, semA0)
        pltpu.sync_copy(ga1, emb_out.at[pl.ds(r1, CHUNK), pl.ds(0, F)])
        pltpu.make_async_copy(dummy, gb1, semB1).wait()
        pltpu.async_copy(h2_hbm.at[dstv.at[pl.ds((j + 2) * CHUNK, CHUNK)]],
                         gb0, semB0)
        pltpu.sync_copy(gb1, emb_out.at[pl.ds(r1, CHUNK), pl.ds(F, F)])

    rl = rbase + (NCHUNK_A - 1) * CHUNK
    pltpu.make_async_copy(dummy, ga0, semA0).wait()
    pltpu.sync_copy(ga0, emb_out.at[pl.ds(rl, CHUNK), pl.ds(0, F)])
    pltpu.make_async_copy(dummy, gb0, semB0).wait()
    pltpu.sync_copy(gb0, emb_out.at[pl.ds(rl, CHUNK), pl.ds(F, F)])

    @pl.loop(0, EPT // LANES)
    def _(i):
        si = srcv[pl.ds(i * LANES, LANES)]
        di = dstv[pl.ds(i * LANES, LANES)]
        vs = plsc.load_gather(sv_v, [si])
        vt = plsc.load_gather(tv_v, [di])
        logv[pl.ds(i * LANES, LANES)] = vs + vt

    pltpu.sync_copy(logv, log_out.at[pl.ds(w * EPT, EPT)])


_gather_call = pl.kernel(
    _gather_body,
    out_type=(
        jax.ShapeDtypeStruct((N_EDGES, 2 * F), jnp.float32),
        jax.ShapeDtypeStruct((N_EDGES,), jnp.float32),
    ),
    mesh=_mesh,
    compiler_params=_cparams,
    scratch_types=[
        pltpu.VMEM((CHUNK, F), jnp.float32),
        pltpu.VMEM((CHUNK, F), jnp.float32),
        pltpu.VMEM((CHUNK, F), jnp.float32),
        pltpu.VMEM((CHUNK, F), jnp.float32),
        pltpu.VMEM((N_NODES,), jnp.float32),
        pltpu.VMEM((N_NODES,), jnp.float32),
        pltpu.VMEM((EPT,), jnp.int32),
        pltpu.VMEM((EPT,), jnp.int32),
        pltpu.VMEM((EPT,), jnp.float32),
        pltpu.SemaphoreType.DMA,
        pltpu.SemaphoreType.DMA,
        pltpu.SemaphoreType.DMA,
        pltpu.SemaphoreType.DMA,
    ],
)


def kernel(edge_attr, edge_index, node_attr, W1, b1, W2, b2, W3, b3):
    src = edge_index[0, :].reshape(-1)
    dst = edge_index[1, :].reshape(-1)
    src3 = src.reshape(NW, NCHUNK_A, CHUNK)
    srcr = src.reshape(NW, EPT)
    dstr = dst.reshape(NW, EPT)
    z128 = jnp.zeros((CHUNK, F), jnp.float32)
    zflat = jnp.zeros((N_NODES,), jnp.float32)

    sums, cnts = _scatter_call(edge_attr, src3, z128, zflat)

    w3r = jnp.concatenate([W3[:F], W3[F:]], axis=1)
    w3r = jnp.pad(w3r, ((0, 0), (0, 6)))
    b3r = jnp.zeros((1, 8), jnp.float32).at[0, 0].set(b3[0])
    h2, st = _dense_call(sums, cnts, node_attr, W1, b1.reshape(1, F),
                         W2, b2.reshape(1, F), w3r, b3r)

    embeddings, logits = _gather_call(h2, srcr, dstr, st[:, 0], st[:, 1])
    return logits, embeddings
